# baseline (speedup 1.0000x reference)
import functools

import jax
import jax.numpy as jnp
from jax import lax
from jax.experimental import pallas as pl
from jax.experimental.pallas import tpu as pltpu
from jax.experimental.pallas import tpu_sc as plsc

NC, NS = 2, 16
V = 1000000
D = 64
B1, B2 = 4096, 200
B = B1 * B2
CPS = D // NC
PPT = B // NS
CB = 1024
NCH = PPT // CB

_MESH = plsc.VectorSubcoreMesh(
    core_axis_name="c", subcore_axis_name="s", num_cores=NC, num_subcores=NS
)


@functools.partial(
    pl.kernel,
    out_type=jax.ShapeDtypeStruct((B2, D, B1), jnp.float32),
    mesh=_MESH,
    compiler_params=pltpu.CompilerParams(use_tc_tiling_on_sc=False),
    scratch_types=[
        pltpu.VMEM((PPT,), jnp.int32),
        pltpu.VMEM((CB,), jnp.float32),
        pltpu.VMEM((CB,), jnp.float32),
        pltpu.VMEM_SHARED((V,), jnp.float32),
        pltpu.SemaphoreType.DMA,
        pltpu.SemaphoreType.DMA,
        pltpu.SemaphoreType.DMA,
    ],
)
def _colgather(xt_hbm, tt_hbm, out_hbm, idx_v, gb0, gb1, colA,
               sg0, sg1, scol):
    cid = lax.axis_index("c")
    sid = lax.axis_index("s")
    p0 = pl.multiple_of(sid * PPT, PPT)
    pltpu.sync_copy(xt_hbm.at[pl.ds(p0, PPT)], idx_v)

    gbufs = (gb0, gb1)
    gsems = (sg0, sg1)
    jbase = cid * CPS

    def load_col(jj, cref):
        pltpu.async_copy(tt_hbm.at[jbase + jj], cref, scol)

    def wait_col(jj, cref):
        pltpu.make_async_copy(tt_hbm.at[jbase + jj], cref, scol).wait()

    def gather_col(jj, cref):
        j = jbase + jj

        def start(k, gb):
            pltpu.async_copy(
                cref.at[idx_v.at[pl.ds(pl.multiple_of(k * CB, CB), CB)]],
                gbufs[gb], gsems[gb]
            )

        def finish(k, gb):
            pltpu.make_async_copy(
                cref.at[idx_v.at[pl.ds(pl.multiple_of(k * CB, CB), CB)]],
                gbufs[gb], gsems[gb]
            ).wait()
            p = p0 + k * CB
            b2 = p >> 12
            b1 = pl.multiple_of(p & (B1 - 1), CB)
            pltpu.sync_copy(gbufs[gb], out_hbm.at[b2, j, pl.ds(b1, CB)])

        start(0, 0)

        @pl.loop(0, NCH, step=2)
        def _chunks(k):
            start(k + 1, 1)
            finish(k, 0)

            @pl.when(k + 2 < NCH)
            def _():
                start(k + 2, 0)

            finish(k + 1, 1)

    @pl.loop(0, CPS)
    def _cols(jj):
        plsc.subcore_barrier()

        @pl.when(sid == 0)
        def _():
            load_col(jj, colA)
            wait_col(jj, colA)

        plsc.subcore_barrier()
        gather_col(jj, colA)


def kernel(x, table):
    xt = x.T.reshape(-1)
    tt = table.T
    out3 = _colgather(xt, tt)
    return jnp.transpose(out3, (2, 0, 1))
